# baseline (speedup 1.0000x reference)
import functools

import jax
import jax.numpy as jnp
from jax import lax
from jax.experimental import pallas as pl
from jax.experimental.pallas import tpu as pltpu
from jax.experimental.pallas import tpu_sc as plsc

_MAX_MASK_PCT = 0.15
_NUM_MASKS = 2
_B, _P, _D = 16, 2048, 1024
_NW = 32
_RPW = _B * _P // _NW
_FILL = 512


def _segment_bounds(X_len):
    rk = jax.random.key(42)
    ka, kb = jax.random.split(rk)
    valid = X_len
    mml = jnp.floor(_MAX_MASK_PCT * valid.astype(jnp.float32)).astype(jnp.int32)
    vrep = jnp.repeat(valid, _NUM_MASKS)
    mrep = jnp.repeat(mml, _NUM_MASKS)
    n = _B * _NUM_MASKS
    t = jnp.floor(jax.random.uniform(ka, (n,)) * (mrep + 1).astype(jnp.float32)).astype(jnp.int32)
    max_start = jnp.clip(vrep - t + 1, 1, None)
    t0 = jnp.floor(jax.random.uniform(kb, (n,)) * max_start.astype(jnp.float32)).astype(jnp.int32)
    t1 = t0 + t
    return jnp.stack(
        [t0.reshape(_B, _NUM_MASKS), t1.reshape(_B, _NUM_MASKS)], axis=-1
    ).reshape(_B, 4)


def _merged_intervals(segs):
    t0a, t1a, t0b, t1b = segs[:, 0], segs[:, 1], segs[:, 2], segs[:, 3]
    first = t0a <= t0b
    s_lo = jnp.where(first, t0a, t0b)
    e_lo = jnp.where(first, t1a, t1b)
    s_hi = jnp.where(first, t0b, t0a)
    e_hi = jnp.where(first, t1b, t1a)
    merged = s_hi <= e_lo
    m1s = s_lo
    m1e = jnp.where(merged, jnp.maximum(e_lo, e_hi), e_lo)
    m2s = jnp.where(merged, _P, s_hi)
    m2e = jnp.where(merged, _P, e_hi)
    empty1 = m1e <= m1s
    m1s = jnp.where(empty1, 0, m1s)
    m1e = jnp.where(empty1, 0, m1e)
    empty2 = m2e <= m2s
    m2s = jnp.where(empty2, _P, m2s)
    m2e = jnp.where(empty2, _P, m2e)
    return jnp.stack([m1s, m1e, m2s, m2e], axis=-1)


_mesh = plsc.VectorSubcoreMesh(core_axis_name="c", subcore_axis_name="s")


@functools.partial(
    pl.kernel,
    mesh=_mesh,
    out_type=jax.ShapeDtypeStruct((_B * _P,), jnp.int32),
    scratch_types=[
        pltpu.VMEM((_RPW,), jnp.int32),
        pltpu.VMEM((16,), jnp.int32),
    ],
)
def _sc_mask_build(segs_hbm, mask_hbm, maskbuf, segs_v):
    wid = lax.axis_index("s") * 2 + lax.axis_index("c")
    base = wid * _RPW
    p0 = (wid % 2) * _RPW

    pltpu.sync_copy(segs_hbm.at[wid], segs_v)
    sv = segs_v[:]
    s0 = sv[0]
    e0 = sv[1]
    s1 = sv[2]
    e1 = sv[3]

    one16 = jnp.full((16,), 1, jnp.int32)
    zero16 = jnp.zeros((16,), jnp.int32)

    def mrow(i, c):
        p = p0 + i * 16 + lax.iota(jnp.int32, 16)
        m = ((p >= s0) & (p < e0)) | ((p >= s1) & (p < e1))
        maskbuf[pl.ds(i * 16, 16)] = jnp.where(m, one16, zero16)
        return c

    lax.fori_loop(0, _RPW // 16, mrow, 0)
    pltpu.sync_copy(maskbuf, mask_hbm.at[pl.ds(base, _RPW)])


def _dma_body(iv_ref, mval_ref, x_ref, o_ref, fill_ref, sem):
    fill_ref[...] = jnp.full((_FILL, 8, 128), mval_ref[0], jnp.float32)

    def batch_body(b, c):
        g = b * _P
        m1s = iv_ref[4 * b]
        m1e = iv_ref[4 * b + 1]
        m2s = iv_ref[4 * b + 2]
        m2e = iv_ref[4 * b + 3]

        def copy_interval(start, end, max_bit):
            n = end - start
            bit = max_bit
            while bit >= 1:
                width = bit
                off = start + (n & ~(2 * width - 1))

                @pl.when((n & width) != 0)
                def _(off=off, width=width):
                    pltpu.async_copy(
                        x_ref.at[pl.ds(g + off, width)],
                        o_ref.at[pl.ds(g + off, width)],
                        sem,
                    )

                bit //= 2

        def fill_interval(start, end, max_bit):
            n = end - start
            bit = max_bit
            while bit >= 1:
                width = bit
                off = start + (n & ~(2 * width - 1))

                @pl.when((n & width) != 0)
                def _(off=off, width=width):
                    pltpu.async_copy(
                        fill_ref.at[pl.ds(0, width)],
                        o_ref.at[pl.ds(g + off, width)],
                        sem,
                    )

                bit //= 2

        copy_interval(0, m1s, 2048)
        fill_interval(m1s, m1e, _FILL)
        copy_interval(m1e, m2s, 2048)
        fill_interval(m2s, m2e, _FILL)
        copy_interval(m2e, _P, 2048)
        return c

    lax.fori_loop(0, _B, batch_body, 0)

    pltpu.make_async_copy(x_ref, o_ref, sem).wait()


_tc_dma_route = pl.pallas_call(
    _dma_body,
    in_specs=[
        pl.BlockSpec(memory_space=pltpu.SMEM),
        pl.BlockSpec(memory_space=pltpu.SMEM),
        pl.BlockSpec(memory_space=pltpu.MemorySpace.HBM),
    ],
    out_specs=pl.BlockSpec(memory_space=pltpu.MemorySpace.HBM),
    out_shape=jax.ShapeDtypeStruct((_B * _P, 8, 128), jnp.float32),
    scratch_shapes=[
        pltpu.VMEM((_FILL, 8, 128), jnp.float32),
        pltpu.SemaphoreType.DMA,
    ],
)


def kernel(X, X_len, mask_value):
    segs = _segment_bounds(X_len)
    segs_w = jnp.repeat(jnp.pad(segs, ((0, 0), (0, 12))), _NW // _B, axis=0)
    mask_i32 = _sc_mask_build(segs_w)
    iv = _merged_intervals(segs).reshape(_B * 4)
    out = _tc_dma_route(iv, mask_value, X.reshape(_B * _P, 8, 128))
    return out.reshape(_B, _P, _D), mask_i32.reshape(_B, _P) != 0


# speedup vs baseline: 31.1395x; 31.1395x over previous
import functools

import jax
import jax.numpy as jnp
from jax import lax
from jax.experimental import pallas as pl
from jax.experimental.pallas import tpu as pltpu
from jax.experimental.pallas import tpu_sc as plsc

_MAX_MASK_PCT = 0.15
_NUM_MASKS = 2
_B, _P, _D = 16, 2048, 1024
_NW = 32
_RPW = _B * _P // _NW
_CH = 32
_NCH = _RPW // _CH


def _segment_bounds(X_len):
    rk = jax.random.key(42)
    ka, kb = jax.random.split(rk)
    valid = X_len
    mml = jnp.floor(_MAX_MASK_PCT * valid.astype(jnp.float32)).astype(jnp.int32)
    vrep = jnp.repeat(valid, _NUM_MASKS)
    mrep = jnp.repeat(mml, _NUM_MASKS)
    n = _B * _NUM_MASKS
    t = jnp.floor(jax.random.uniform(ka, (n,)) * (mrep + 1).astype(jnp.float32)).astype(jnp.int32)
    max_start = jnp.clip(vrep - t + 1, 1, None)
    t0 = jnp.floor(jax.random.uniform(kb, (n,)) * max_start.astype(jnp.float32)).astype(jnp.int32)
    t1 = t0 + t
    return jnp.stack(
        [t0.reshape(_B, _NUM_MASKS), t1.reshape(_B, _NUM_MASKS)], axis=-1
    ).reshape(_B, 4)


_mesh = plsc.VectorSubcoreMesh(core_axis_name="c", subcore_axis_name="s")


@functools.partial(
    pl.kernel,
    mesh=_mesh,
    out_type=[
        jax.ShapeDtypeStruct((_B * _P, _D), jnp.float32),
        jax.ShapeDtypeStruct((_B * _P,), jnp.int32),
    ],
    scratch_types=[
        pltpu.VMEM((_CH, _D), jnp.float32),
        pltpu.VMEM((_CH, _D), jnp.float32),
        pltpu.VMEM((_CH, _D), jnp.float32),
        pltpu.VMEM((_RPW,), jnp.int32),
        pltpu.VMEM((16,), jnp.int32),
        pltpu.VMEM((16,), jnp.float32),
        pltpu.SemaphoreType.DMA,
        pltpu.SemaphoreType.DMA,
        pltpu.SemaphoreType.DMA,
        pltpu.SemaphoreType.DMA,
    ],
)
def _sc_stream_copy(x_hbm, segs_hbm, mval_hbm, out_hbm, mask_hbm,
                    b0, b1, mvchunk, maskbuf, segs_v, mval_v,
                    isem0, isem1, osem0, osem1):
    bufs = (b0, b1)
    isems = (isem0, isem1)
    osems = (osem0, osem1)

    wid = lax.axis_index("s") * 2 + lax.axis_index("c")
    base = wid * _RPW
    p0 = (wid % 2) * _RPW

    pltpu.sync_copy(segs_hbm.at[wid], segs_v)
    pltpu.sync_copy(mval_hbm, mval_v)
    sv = segs_v[:]
    s0 = sv[0]
    e0 = sv[1]
    s1 = sv[2]
    e1 = sv[3]
    mv = mval_v[:]

    def fillrow(i, c):
        for cc in range(_D // 16):
            mvchunk[i, pl.ds(16 * cc, 16)] = mv
        return c

    lax.fori_loop(0, _CH, fillrow, 0)

    one16 = jnp.full((16,), 1, jnp.int32)
    zero16 = jnp.zeros((16,), jnp.int32)

    def mrow(i, c):
        p = p0 + i * 16 + lax.iota(jnp.int32, 16)
        m = ((p >= s0) & (p < e0)) | ((p >= s1) & (p < e1))
        maskbuf[pl.ds(i * 16, 16)] = jnp.where(m, one16, zero16)
        return c

    lax.fori_loop(0, _RPW // 16, mrow, 0)
    pltpu.sync_copy(maskbuf, mask_hbm.at[pl.ds(base, _RPW)])

    pltpu.async_copy(x_hbm.at[pl.ds(base, _CH)], bufs[0], isems[0])
    for i in range(_NCH):
        k = i % 2
        nk = 1 - k
        r0 = base + i * _CH
        if i + 1 < _NCH:
            if i >= 1:
                pltpu.make_async_copy(
                    bufs[nk], out_hbm.at[pl.ds(r0 - _CH, _CH)], osems[nk]
                ).wait()
            pltpu.async_copy(
                x_hbm.at[pl.ds(r0 + _CH, _CH)], bufs[nk], isems[nk]
            )
        pltpu.make_async_copy(
            x_hbm.at[pl.ds(r0, _CH)], bufs[k], isems[k]
        ).wait()

        lo = p0 + i * _CH
        hi = lo + _CH
        inside = ((lo >= s0) & (hi <= e0)) | ((lo >= s1) & (hi <= e1))
        clear0 = (hi <= s0) | (lo >= e0) | (e0 <= s0)
        clear1 = (hi <= s1) | (lo >= e1) | (e1 <= s1)
        untouched = clear0 & clear1
        mixed = jnp.logical_not(untouched | inside)

        @pl.when(mixed)
        def _(lo=lo, k=k):
            def row(j, c):
                p = lo + j
                masked = ((p >= s0) & (p < e0)) | ((p >= s1) & (p < e1))

                @pl.when(masked)
                def _():
                    for cc in range(_D // 16):
                        bufs[k][j, pl.ds(16 * cc, 16)] = mv

                return c

            lax.fori_loop(0, _CH, row, 0)

        @pl.when(inside)
        def _(r0=r0, k=k):
            pltpu.async_copy(mvchunk, out_hbm.at[pl.ds(r0, _CH)], osems[k])

        @pl.when(jnp.logical_not(inside))
        def _(r0=r0, k=k):
            pltpu.async_copy(bufs[k], out_hbm.at[pl.ds(r0, _CH)], osems[k])

    pltpu.make_async_copy(
        bufs[(_NCH - 2) % 2], out_hbm.at[pl.ds(base + (_NCH - 2) * _CH, _CH)],
        osems[(_NCH - 2) % 2],
    ).wait()
    pltpu.make_async_copy(
        bufs[(_NCH - 1) % 2], out_hbm.at[pl.ds(base + (_NCH - 1) * _CH, _CH)],
        osems[(_NCH - 1) % 2],
    ).wait()


def kernel(X, X_len, mask_value):
    segs = _segment_bounds(X_len)
    segs_w = jnp.repeat(jnp.pad(segs, ((0, 0), (0, 12))), _NW // _B, axis=0)
    mval16 = jnp.full((16,), mask_value[0], jnp.float32)
    out, mask_i32 = _sc_stream_copy(X.reshape(_B * _P, _D), segs_w, mval16)
    return out.reshape(_B, _P, _D), mask_i32.reshape(_B, _P) != 0
